# baseline (speedup 1.0000x reference)
import dataclasses
import functools

import jax
import jax.numpy as jnp
from jax import lax
from jax.experimental import pallas as pl
from jax.experimental.pallas import tpu as pltpu
from jax.experimental.pallas import tpu_sc as plsc

N = 10000
E = 160000
D = 256
DP = 128
OUT = 256

NC = 2
NS = 16
NW = NC * NS

R = 320
MR = 328
DEGR = 336
NPAD = NW * R
SC_ROWS = NS * R
SC_PAD = 5128

C = 3200
NCHUNK = E // C
G = 96
CPAD = C + 2 * G

FMIN = -3.4028235e38
PADV = 1 << 20


def _sc_segment_kernel(xlo_hbm, xhi_hbm, src_hbm, dst_hbm,
                       slo_hbm, shi_hbm, mlo_hbm, mhi_hbm, deg_hbm,
                       srcv, dstv, gsrc, gdst, gix0, gix1, gsc0, gsc1,
                       rowbuf0, rowbuf1, maxacc, degv, spacc,
                       sem_e, sem_g0, sem_g1, sem_a):
  c = lax.axis_index("c")
  s = lax.axis_index("s")
  g = c * NS + s
  base = g * R
  my_r = jnp.minimum(R, N - base)
  s_off = s * R

  lane_iota = lax.iota(jnp.int32, 16)
  onesf = jnp.full((16,), 1.0, jnp.float32)
  zerof = jnp.full((16,), 0.0, jnp.float32)
  fminf = jnp.full((16,), FMIN, jnp.float32)

  gixs = (gix0, gix1)
  gscs = (gsc0, gsc1)
  bufs = (rowbuf0, rowbuf1)
  gsems = (sem_g0, sem_g1)

  @pl.loop(0, DEGR // 16)
  def _(i):
    degv[pl.ds(16 * i, 16)] = zerof

  def stage(b, q, count_deg):
    off = b * G
    for t in range(G // 16):
      sv = gsrc[pl.ds(off + 16 * t, 16)]
      dv = gdst[pl.ds(off + 16 * t, 16)]
      gixs[q][pl.ds(16 * t, 16)] = sv
      gscs[q][pl.ds(16 * t, 16)] = jnp.minimum(dv + s_off, SC_PAD - 1)
      if count_deg:
        plsc.addupdate_scatter(degv, [jnp.minimum(dv, DEGR - 1)], onesf,
                               mask=dv < R)

  for half, (x_hbm, sum_hbm, max_hbm) in enumerate((
      (xlo_hbm, slo_hbm, mlo_hbm),
      (xhi_hbm, shi_hbm, mhi_hbm),
  )):
    first = half == 0

    @pl.loop(0, MR)
    def _(r):
      for j in range(DP // 16):
        maxacc[r, pl.ds(16 * j, 16)] = fminf

    @pl.loop(0, G)
    def _(r):
      for j in range(DP // 16):
        rowbuf0[r, pl.ds(16 * j, 16)] = zerof

    for blk in range(0, R, 64):
      pltpu.sync_copy(rowbuf0.at[pl.ds(0, 64)],
                      spacc.at[pl.ds(s_off + blk, 64)])

    pltpu.async_copy(src_hbm.at[pl.ds(0, C)], srcv, sem_e)
    pltpu.async_copy(dst_hbm.at[pl.ds(0, C)], dstv, sem_e)

    @pl.loop(0, NCHUNK)
    def _(k):
      pltpu.make_async_copy(src_hbm.at[pl.ds(0, C)], srcv, sem_e).wait()
      pltpu.make_async_copy(dst_hbm.at[pl.ds(0, C)], dstv, sem_e).wait()

      def cbody(i, cnt):
        d16 = dstv[pl.ds(16 * i, 16)]
        s16 = srcv[pl.ds(16 * i, 16)]
        dloc = d16 - base
        m = (dloc >= 0) & (dloc < my_r)
        plsc.store_compressed(gdst.at[pl.ds(cnt, 16)], dloc, mask=m)
        plsc.store_compressed(gsrc.at[pl.ds(cnt, 16)], s16, mask=m)
        return cnt + jnp.sum(jnp.where(m, 1, 0))

      cnt = lax.fori_loop(0, C // 16, cbody, jnp.int32(0))

      @pl.when(k + 1 < NCHUNK)
      def _():
        e1 = (k + 1) * C
        pltpu.async_copy(src_hbm.at[pl.ds(e1, C)], srcv, sem_e)
        pltpu.async_copy(dst_hbm.at[pl.ds(e1, C)], dstv, sem_e)

      pad_src = jnp.full((16,), 0, jnp.int32)
      pad_dst = jnp.full((16,), PADV, jnp.int32)
      for t in range(G // 16):
        gsrc[pl.ds(cnt + 16 * t, 16)] = pad_src
        gdst[pl.ds(cnt + 16 * t, 16)] = pad_dst

      nb = (cnt + (G - 1)) // G

      @pl.when(nb > 0)
      def _():
        stage(0, 0, first)
        pltpu.async_copy(x_hbm.at[gix0], rowbuf0, sem_g0)

      def pairbody(p, carry):
        for q in (0, 1):
          b = 2 * p + q

          @pl.when(b < nb)
          def _():
            pltpu.make_async_copy(x_hbm.at[gixs[q]], bufs[q],
                                  gsems[q]).wait()

            @pl.when(b + 1 < nb)
            def _():
              stage(b + 1, 1 - q, first)
              pltpu.async_copy(x_hbm.at[gixs[1 - q]], bufs[1 - q],
                               gsems[1 - q])

            pltpu.async_copy(bufs[q], spacc.at[gscs[q]], sem_a, add=True)

            @pl.loop(0, G, unroll=2)
            def _(e):
              e16 = (e // 16) * 16
              dv = gdst[pl.ds(b * G + e16, 16)]
              lane = e - e16
              dscal = jnp.sum(jnp.where(lane_iota == lane, dv, 0))
              dscal = jnp.minimum(dscal, MR - 1)
              for j in range(DP // 16):
                bv = bufs[q][e, pl.ds(16 * j, 16)]
                mv = maxacc[dscal, pl.ds(16 * j, 16)]
                maxacc[dscal, pl.ds(16 * j, 16)] = jnp.maximum(mv, bv)

            pltpu.make_async_copy(bufs[q], spacc.at[gscs[q]], sem_a).wait()

        return carry

      lax.fori_loop(0, (nb + 1) // 2, pairbody, jnp.int32(0))

    pltpu.sync_copy(spacc.at[pl.ds(s_off, R)], sum_hbm.at[pl.ds(base, R)])
    pltpu.sync_copy(maxacc.at[pl.ds(0, R)], max_hbm.at[pl.ds(base, R)])

  pltpu.sync_copy(degv.at[pl.ds(0, R)], deg_hbm.at[pl.ds(base, R)])


@jax.jit
def _sc_segment(xlo, xhi, src, dst):
  mesh = plsc.VectorSubcoreMesh(core_axis_name="c", subcore_axis_name="s")
  cp = pltpu.CompilerParams()
  if "needs_layout_passes" in pltpu.CompilerParams.__dataclass_fields__:
    cp = dataclasses.replace(cp, needs_layout_passes=False)
  f = pl.kernel(
      _sc_segment_kernel,
      mesh=mesh,
      compiler_params=cp,
      out_type=[
          jax.ShapeDtypeStruct((NPAD, DP), jnp.float32),
          jax.ShapeDtypeStruct((NPAD, DP), jnp.float32),
          jax.ShapeDtypeStruct((NPAD, DP), jnp.float32),
          jax.ShapeDtypeStruct((NPAD, DP), jnp.float32),
          jax.ShapeDtypeStruct((NPAD,), jnp.float32),
      ],
      scratch_types=[
          pltpu.VMEM((C,), jnp.int32),
          pltpu.VMEM((C,), jnp.int32),
          pltpu.VMEM((CPAD,), jnp.int32),
          pltpu.VMEM((CPAD,), jnp.int32),
          pltpu.VMEM((G,), jnp.int32),
          pltpu.VMEM((G,), jnp.int32),
          pltpu.VMEM((G,), jnp.int32),
          pltpu.VMEM((G,), jnp.int32),
          pltpu.VMEM((G, DP), jnp.float32),
          pltpu.VMEM((G, DP), jnp.float32),
          pltpu.VMEM((MR, DP), jnp.float32),
          pltpu.VMEM((DEGR,), jnp.float32),
          pltpu.VMEM_SHARED((SC_PAD, DP), jnp.float32),
          pltpu.SemaphoreType.DMA,
          pltpu.SemaphoreType.DMA,
          pltpu.SemaphoreType.DMA,
          pltpu.SemaphoreType.DMA,
      ],
  )
  return f(xlo, xhi, src, dst)


BN = 2560


def _linear_kernel(x_ref, mlo_ref, mhi_ref, slo_ref, shi_ref,
                   w1_ref, w2lo_ref, w2hi_ref, w3lo_ref, w3hi_ref,
                   b_ref, o_ref):
  acc = jnp.dot(x_ref[...], w1_ref[...], preferred_element_type=jnp.float32)
  for lhs, w in ((mlo_ref, w2lo_ref), (mhi_ref, w2hi_ref),
                 (slo_ref, w3lo_ref), (shi_ref, w3hi_ref)):
    acc = acc + jnp.dot(lhs[...], w[...], preferred_element_type=jnp.float32)
  acc = acc + b_ref[...]
  o_ref[...] = jnp.maximum(acc, 0.0)


@jax.jit
def _tc_linear(xp, mlo, mhi, slo, shi, w1t, w2lo, w2hi, w3lo, w3hi, b2):
  grid = NPAD // BN
  row_spec = lambda nc: pl.BlockSpec((BN, nc), lambda i: (i, 0))
  w_spec = lambda nr: pl.BlockSpec((nr, OUT), lambda i: (0, 0))
  return pl.pallas_call(
      _linear_kernel,
      grid=(grid,),
      in_specs=[
          row_spec(D), row_spec(DP), row_spec(DP), row_spec(DP), row_spec(DP),
          w_spec(D), w_spec(DP), w_spec(DP), w_spec(DP), w_spec(DP),
          pl.BlockSpec((1, OUT), lambda i: (0, 0)),
      ],
      out_specs=pl.BlockSpec((BN, OUT), lambda i: (i, 0)),
      out_shape=jax.ShapeDtypeStruct((NPAD, OUT), jnp.float32),
  )(xp, mlo, mhi, slo, shi, w1t, w2lo, w2hi, w3lo, w3hi, b2)


def kernel(x, edge_index, W, b):
  src = edge_index[0]
  dst = edge_index[1]

  xlo = x[:, :DP]
  xhi = x[:, DP:]
  slo, shi, mlo, mhi, deg = _sc_segment(xlo, xhi, src, dst)

  inv = (1.0 / jnp.maximum(deg, 1.0))[:, None]
  nz = (deg > 0.0)[:, None]
  mean_lo = slo * inv
  mean_hi = shi * inv
  smax_lo = jnp.where(nz, mlo, 0.0)
  smax_hi = jnp.where(nz, mhi, 0.0)

  xp = jnp.zeros((NPAD, D), jnp.float32).at[:N].set(x)
  w1t = W[:, :D].T
  w2lo = W[:, D:D + DP].T
  w2hi = W[:, D + DP:2 * D].T
  w3lo = W[:, 2 * D:2 * D + DP].T
  w3hi = W[:, 2 * D + DP:].T
  b2 = b.reshape(1, OUT)

  h = _tc_linear(xp, mean_lo, mean_hi, smax_lo, smax_hi,
                 w1t, w2lo, w2hi, w3lo, w3hi, b2)
  return h[:N]


# speedup vs baseline: 2.1651x; 2.1651x over previous
import dataclasses
import functools

import jax
import jax.numpy as jnp
from jax import lax
from jax.experimental import pallas as pl
from jax.experimental.pallas import tpu as pltpu
from jax.experimental.pallas import tpu_sc as plsc

N = 10000
E = 160000
D = 256
DP = 128
OUT = 256

NC = 2
NS = 16
NW = NC * NS

R = 320
MR = 328
DEGR = 336
NPAD = NW * R

C = 3200
NCHUNK = E // C
G = 48
CPAD = C + 2 * G

FMIN = -3.4028235e38
PADV = 1 << 20


def _sc_segment_kernel(xlo_hbm, xhi_hbm, src_hbm, dst_hbm,
                       slo_hbm, shi_hbm, mlo_hbm, mhi_hbm, deg_hbm,
                       srcv, dstv, gsrc, gdst, gix0, gix1,
                       rowbuf0, rowbuf1, maxacc, sumacc, degv,
                       sem_e, sem_g0, sem_g1):
  c = lax.axis_index("c")
  s = lax.axis_index("s")
  g = c * NS + s
  base = g * R
  my_r = jnp.minimum(R, N - base)

  lane_iota = lax.iota(jnp.int32, 16)
  onesf = jnp.full((16,), 1.0, jnp.float32)
  zerof = jnp.full((16,), 0.0, jnp.float32)
  fminf = jnp.full((16,), FMIN, jnp.float32)

  gixs = (gix0, gix1)
  bufs = (rowbuf0, rowbuf1)
  gsems = (sem_g0, sem_g1)

  @pl.loop(0, DEGR // 16)
  def _(i):
    degv[pl.ds(16 * i, 16)] = zerof

  def stage(b, q, count_deg):
    off = b * G
    for t in range(G // 16):
      sv = gsrc[pl.ds(off + 16 * t, 16)]
      gixs[q][pl.ds(16 * t, 16)] = sv
      if count_deg:
        dv = gdst[pl.ds(off + 16 * t, 16)]
        plsc.addupdate_scatter(degv, [jnp.minimum(dv, DEGR - 1)], onesf,
                               mask=dv < R)

  for half, (x_hbm, sum_hbm, max_hbm) in enumerate((
      (xlo_hbm, slo_hbm, mlo_hbm),
      (xhi_hbm, shi_hbm, mhi_hbm),
  )):
    first = half == 0

    @pl.loop(0, MR)
    def _(r):
      for j in range(DP // 16):
        maxacc[r, pl.ds(16 * j, 16)] = fminf
        sumacc[r, pl.ds(16 * j, 16)] = zerof

    pltpu.async_copy(src_hbm.at[pl.ds(0, C)], srcv, sem_e)
    pltpu.async_copy(dst_hbm.at[pl.ds(0, C)], dstv, sem_e)

    @pl.loop(0, NCHUNK)
    def _(k):
      pltpu.make_async_copy(src_hbm.at[pl.ds(0, C)], srcv, sem_e).wait()
      pltpu.make_async_copy(dst_hbm.at[pl.ds(0, C)], dstv, sem_e).wait()

      def cbody(i, cnt):
        d16 = dstv[pl.ds(16 * i, 16)]
        s16 = srcv[pl.ds(16 * i, 16)]
        dloc = d16 - base
        m = (dloc >= 0) & (dloc < my_r)
        plsc.store_compressed(gdst.at[pl.ds(cnt, 16)], dloc, mask=m)
        plsc.store_compressed(gsrc.at[pl.ds(cnt, 16)], s16, mask=m)
        return cnt + jnp.sum(jnp.where(m, 1, 0))

      cnt = lax.fori_loop(0, C // 16, cbody, jnp.int32(0))

      @pl.when(k + 1 < NCHUNK)
      def _():
        e1 = (k + 1) * C
        pltpu.async_copy(src_hbm.at[pl.ds(e1, C)], srcv, sem_e)
        pltpu.async_copy(dst_hbm.at[pl.ds(e1, C)], dstv, sem_e)

      pad_src = jnp.full((16,), 0, jnp.int32)
      pad_dst = jnp.full((16,), PADV, jnp.int32)
      for t in range(G // 16):
        gsrc[pl.ds(cnt + 16 * t, 16)] = pad_src
        gdst[pl.ds(cnt + 16 * t, 16)] = pad_dst

      nb = (cnt + (G - 1)) // G

      @pl.when(nb > 0)
      def _():
        stage(0, 0, first)
        pltpu.async_copy(x_hbm.at[gix0], rowbuf0, sem_g0)

      def pairbody(p, carry):
        for q in (0, 1):
          b = 2 * p + q

          @pl.when(b < nb)
          def _():
            pltpu.make_async_copy(x_hbm.at[gixs[q]], bufs[q],
                                  gsems[q]).wait()

            @pl.when(b + 1 < nb)
            def _():
              stage(b + 1, 1 - q, first)
              pltpu.async_copy(x_hbm.at[gixs[1 - q]], bufs[1 - q],
                               gsems[1 - q])

            @pl.loop(0, G, unroll=2)
            def _(e):
              e16 = (e // 16) * 16
              dv = gdst[pl.ds(b * G + e16, 16)]
              lane = e - e16
              dscal = jnp.sum(jnp.where(lane_iota == lane, dv, 0))
              dscal = jnp.minimum(dscal, MR - 1)
              for j in range(DP // 16):
                bv = bufs[q][e, pl.ds(16 * j, 16)]
                mv = maxacc[dscal, pl.ds(16 * j, 16)]
                maxacc[dscal, pl.ds(16 * j, 16)] = jnp.maximum(mv, bv)
                sv = sumacc[dscal, pl.ds(16 * j, 16)]
                sumacc[dscal, pl.ds(16 * j, 16)] = sv + bv

        return carry

      lax.fori_loop(0, (nb + 1) // 2, pairbody, jnp.int32(0))

    pltpu.sync_copy(sumacc.at[pl.ds(0, R)], sum_hbm.at[pl.ds(base, R)])
    pltpu.sync_copy(maxacc.at[pl.ds(0, R)], max_hbm.at[pl.ds(base, R)])

  pltpu.sync_copy(degv.at[pl.ds(0, R)], deg_hbm.at[pl.ds(base, R)])


@jax.jit
def _sc_segment(xlo, xhi, src, dst):
  mesh = plsc.VectorSubcoreMesh(core_axis_name="c", subcore_axis_name="s")
  cp = pltpu.CompilerParams()
  if "needs_layout_passes" in pltpu.CompilerParams.__dataclass_fields__:
    cp = dataclasses.replace(cp, needs_layout_passes=False)
  f = pl.kernel(
      _sc_segment_kernel,
      mesh=mesh,
      compiler_params=cp,
      out_type=[
          jax.ShapeDtypeStruct((NPAD, DP), jnp.float32),
          jax.ShapeDtypeStruct((NPAD, DP), jnp.float32),
          jax.ShapeDtypeStruct((NPAD, DP), jnp.float32),
          jax.ShapeDtypeStruct((NPAD, DP), jnp.float32),
          jax.ShapeDtypeStruct((NPAD,), jnp.float32),
      ],
      scratch_types=[
          pltpu.VMEM((C,), jnp.int32),
          pltpu.VMEM((C,), jnp.int32),
          pltpu.VMEM((CPAD,), jnp.int32),
          pltpu.VMEM((CPAD,), jnp.int32),
          pltpu.VMEM((G,), jnp.int32),
          pltpu.VMEM((G,), jnp.int32),
          pltpu.VMEM((G, DP), jnp.float32),
          pltpu.VMEM((G, DP), jnp.float32),
          pltpu.VMEM((MR, DP), jnp.float32),
          pltpu.VMEM((MR, DP), jnp.float32),
          pltpu.VMEM((DEGR,), jnp.float32),
          pltpu.SemaphoreType.DMA,
          pltpu.SemaphoreType.DMA,
          pltpu.SemaphoreType.DMA,
      ],
  )
  return f(xlo, xhi, src, dst)


BN = 2560


def _linear_kernel(x_ref, mlo_ref, mhi_ref, slo_ref, shi_ref,
                   w1_ref, w2lo_ref, w2hi_ref, w3lo_ref, w3hi_ref,
                   b_ref, o_ref):
  acc = jnp.dot(x_ref[...], w1_ref[...], preferred_element_type=jnp.float32)
  for lhs, w in ((mlo_ref, w2lo_ref), (mhi_ref, w2hi_ref),
                 (slo_ref, w3lo_ref), (shi_ref, w3hi_ref)):
    acc = acc + jnp.dot(lhs[...], w[...], preferred_element_type=jnp.float32)
  acc = acc + b_ref[...]
  o_ref[...] = jnp.maximum(acc, 0.0)


@jax.jit
def _tc_linear(xp, mlo, mhi, slo, shi, w1t, w2lo, w2hi, w3lo, w3hi, b2):
  grid = NPAD // BN
  row_spec = lambda nc: pl.BlockSpec((BN, nc), lambda i: (i, 0))
  w_spec = lambda nr: pl.BlockSpec((nr, OUT), lambda i: (0, 0))
  return pl.pallas_call(
      _linear_kernel,
      grid=(grid,),
      in_specs=[
          row_spec(D), row_spec(DP), row_spec(DP), row_spec(DP), row_spec(DP),
          w_spec(D), w_spec(DP), w_spec(DP), w_spec(DP), w_spec(DP),
          pl.BlockSpec((1, OUT), lambda i: (0, 0)),
      ],
      out_specs=pl.BlockSpec((BN, OUT), lambda i: (i, 0)),
      out_shape=jax.ShapeDtypeStruct((NPAD, OUT), jnp.float32),
  )(xp, mlo, mhi, slo, shi, w1t, w2lo, w2hi, w3lo, w3hi, b2)


def kernel(x, edge_index, W, b):
  src = edge_index[0]
  dst = edge_index[1]

  xlo = x[:, :DP]
  xhi = x[:, DP:]
  slo, shi, mlo, mhi, deg = _sc_segment(xlo, xhi, src, dst)

  inv = (1.0 / jnp.maximum(deg, 1.0))[:, None]
  nz = (deg > 0.0)[:, None]
  mean_lo = slo * inv
  mean_hi = shi * inv
  smax_lo = jnp.where(nz, mlo, 0.0)
  smax_hi = jnp.where(nz, mhi, 0.0)

  xp = jnp.zeros((NPAD, D), jnp.float32).at[:N].set(x)
  w1t = W[:, :D].T
  w2lo = W[:, D:D + DP].T
  w2hi = W[:, D + DP:2 * D].T
  w3lo = W[:, 2 * D:2 * D + DP].T
  w3hi = W[:, 2 * D + DP:].T
  b2 = b.reshape(1, OUT)

  h = _tc_linear(xp, mean_lo, mean_hi, smax_lo, smax_hi,
                 w1t, w2lo, w2hi, w3lo, w3hi, b2)
  return h[:N]
